# baseline (speedup 1.0000x reference)
import functools
import jax
import jax.numpy as jnp
from jax import lax
from jax.experimental import pallas as pl
from jax.experimental.pallas import tpu as pltpu
from jax.experimental.pallas import tpu_sc as plsc

B, KP1, D, V = 1024, 512, 128, 100000
N = B * KP1
T = 0.07

NC, NS = 2, 16
NW = NC * NS
CHUNK = 128
PER_W = N // NW
STEPS = PER_W // CHUNK
BPW = B // NW
CPB = KP1 // CHUNK
L = 16


@functools.lru_cache(maxsize=None)
def _make_sc_fused():
    mesh = plsc.VectorSubcoreMesh(
        core_axis_name="c", subcore_axis_name="s",
        num_cores=NC, num_subcores=NS)

    @functools.partial(
        pl.kernel,
        out_type=[jax.ShapeDtypeStruct((N,), jnp.float32),
                  jax.ShapeDtypeStruct((N,), jnp.float32)],
        mesh=mesh,
        compiler_params=pltpu.CompilerParams(needs_layout_passes=False),
        scratch_types=[
            pltpu.VMEM((STEPS, CHUNK), jnp.int32),
            pltpu.VMEM((CHUNK, D), jnp.float32),
            pltpu.VMEM((CHUNK, D), jnp.float32),
            pltpu.VMEM((CHUNK, D), jnp.float32),
            pltpu.VMEM((CHUNK, D), jnp.float32),
            pltpu.VMEM((BPW, D), jnp.float32),
            pltpu.VMEM((BPW, D), jnp.float32),
            pltpu.VMEM((PER_W,), jnp.float32),
            pltpu.VMEM((PER_W,), jnp.float32),
            pltpu.VMEM((L * 17,), jnp.float32),
            pltpu.VMEM((L * 17,), jnp.float32),
            pltpu.SemaphoreType.DMA,
            pltpu.SemaphoreType.DMA,
        ],
    )
    def _sc_fused(t0_hbm, t1_hbm, idx_hbm, e1_hbm, e0_hbm,
                  o0_hbm, o1_hbm,
                  idx_v, r00, r01, r10, r11, e1_v, e0_v,
                  out0_v, out1_v, tsc0, tsc1, g0, g1):
        wid = lax.axis_index("s") * NC + lax.axis_index("c")
        pltpu.sync_copy(idx_hbm.at[wid], idx_v)
        pltpu.sync_copy(e1_hbm.at[pl.ds(wid * BPW, BPW)], e1_v)
        pltpu.sync_copy(e0_hbm.at[pl.ds(wid * BPW, BPW)], e0_v)

        iota = lax.iota(jnp.int32, L)
        base17 = iota * 17

        rows_bufs = ((r00, r01), (r10, r11))
        gsems = (g0, g1)

        def issue(j, par):
            pltpu.async_copy(t0_hbm.at[idx_v.at[j]], rows_bufs[par][0],
                             gsems[par])
            pltpu.async_copy(t1_hbm.at[idx_v.at[j]], rows_bufs[par][1],
                             gsems[par])

        def drain(j, par):
            pltpu.make_async_copy(t0_hbm.at[idx_v.at[j]], rows_bufs[par][0],
                                  gsems[par]).wait()
            pltpu.make_async_copy(t1_hbm.at[idx_v.at[j]], rows_bufs[par][1],
                                  gsems[par]).wait()

        issue(0, 0)

        def compute(j, par):
            bl = j // CPB
            NCH = D // L
            tabs = ((rows_bufs[par][0], e1_v, out0_v, tsc0),
                    (rows_bufs[par][1], e0_v, out1_v, tsc1))
            evs = [[e_v[bl, pl.ds(16 * c, L)] for c in range(NCH)]
                   for (_, e_v, _, _) in tabs]

            def grp(g, carry):
                r0 = g * L
                for tbl, (rows, _, _, tsc) in enumerate(tabs):
                    ev = evs[tbl]
                    for l in range(L):
                        r = r0 + l
                        a = rows[r, pl.ds(0, L)] * ev[0]
                        b = rows[r, pl.ds(16, L)] * ev[1]
                        for c in range(2, NCH, 2):
                            a = a + rows[r, pl.ds(16 * c, L)] * ev[c]
                            b = b + rows[r, pl.ds(16 * (c + 1), L)] * ev[c + 1]
                        plsc.store_scatter(tsc, [base17 + l], a + b)
                for tbl, (_, _, out_v, tsc) in enumerate(tabs):
                    cols = [tsc[pl.ds(17 * c, L)] for c in range(L)]
                    while len(cols) > 1:
                        cols = [cols[k] + cols[k + 1]
                                for k in range(0, len(cols), 2)]
                    out_v[pl.ds(j * CHUNK + r0, L)] = cols[0]
                return carry

            lax.fori_loop(0, CHUNK // L, grp, 0, unroll=False)

        def body(i, carry):
            for par in (0, 1):
                j = 2 * i + par

                @pl.when(j + 1 < STEPS)
                def _():
                    issue(j + 1, 1 - par)

                drain(j, par)
                compute(j, par)
            return carry

        lax.fori_loop(0, STEPS // 2, body, 0, unroll=False)

        base = wid * PER_W
        pltpu.sync_copy(out0_v, o0_hbm.at[pl.ds(base, PER_W)])
        pltpu.sync_copy(out1_v, o1_hbm.at[pl.ds(base, PER_W)])

    return _sc_fused


BB = 8
GRID = B // BB


def _tc_exp_body(s0_ref, s1_ref, o0_ref, o1_ref, p0_ref, p1_ref):
    o0 = jnp.exp(s0_ref[...] * (1.0 / T))
    o1 = jnp.exp(s1_ref[...] * (1.0 / T))
    o0_ref[...] = o0
    o1_ref[...] = o1
    p0_ref[...] = jnp.sum(o0, axis=1).reshape(1, 1, BB)
    p1_ref[...] = jnp.sum(o1, axis=1).reshape(1, 1, BB)


_tc_exp = pl.pallas_call(
    _tc_exp_body,
    grid=(GRID,),
    in_specs=[
        pl.BlockSpec((BB, KP1), lambda i: (i, 0)),
        pl.BlockSpec((BB, KP1), lambda i: (i, 0)),
    ],
    out_specs=[
        pl.BlockSpec((BB, KP1), lambda i: (i, 0)),
        pl.BlockSpec((BB, KP1), lambda i: (i, 0)),
        pl.BlockSpec((1, 1, BB), lambda i: (i, 0, 0)),
        pl.BlockSpec((1, 1, BB), lambda i: (i, 0, 0)),
    ],
    out_shape=[
        jax.ShapeDtypeStruct((B, KP1), jnp.float32),
        jax.ShapeDtypeStruct((B, KP1), jnp.float32),
        jax.ShapeDtypeStruct((GRID, 1, BB), jnp.float32),
        jax.ShapeDtypeStruct((GRID, 1, BB), jnp.float32),
    ],
)


def _tc_scale_body(sc_ref, o0_ref, o1_ref, r0_ref, r1_ref):
    r0_ref[...] = o0_ref[...] * sc_ref[0]
    r1_ref[...] = o1_ref[...] * sc_ref[1]


_tc_scale = pl.pallas_call(
    _tc_scale_body,
    in_specs=[
        pl.BlockSpec(memory_space=pltpu.SMEM),
        pl.BlockSpec((B, KP1), lambda: (0, 0)),
        pl.BlockSpec((B, KP1), lambda: (0, 0)),
    ],
    out_specs=[
        pl.BlockSpec((B, KP1), lambda: (0, 0)),
        pl.BlockSpec((B, KP1), lambda: (0, 0)),
    ],
    out_shape=[
        jax.ShapeDtypeStruct((B, KP1), jnp.float32),
        jax.ShapeDtypeStruct((B, KP1), jnp.float32),
    ],
)


def kernel(embedings, y, idx, memory_v0, memory_v1):
    idx3 = idx.reshape(NW, STEPS, CHUNK)
    s0, s1 = _make_sc_fused()(
        memory_v0, memory_v1, idx3, embedings[1], embedings[0])
    o0, o1, ps0, ps1 = _tc_exp(s0.reshape(B, KP1), s1.reshape(B, KP1))
    scale = jnp.stack([1.0 / (jnp.sum(ps0) / N * V),
                       1.0 / (jnp.sum(ps1) / N * V)])
    r0, r1 = _tc_scale(scale, o0, o1)
    return (r0[:, :, None], r1[:, :, None])
